# Initial kernel scaffold; baseline (speedup 1.0000x reference)
import functools
import math

import jax
import jax.numpy as jnp
from jax import lax
from jax.experimental import pallas as pl
from jax.experimental.pallas import tpu as pltpu
from jax.experimental.pallas import tpu_sc as plsc

N = 10000
D = 128
D_OUT = 40
ALPHA = 0.1
THETA = 0.5
EPS = 1e-5

NC = 2
NS = 16
CHUNK = 128
N_PAD = 10240
ZROWS = N_PAD // NS
DROWS = N // NS


def _mesh():
    return plsc.VectorSubcoreMesh(core_axis_name="c", subcore_axis_name="s")


def _zero_buf(buf, rows, width):
    def body(i, carry):
        for l in range(width // 16):
            buf[i, pl.ds(l * 16, 16)] = jnp.zeros((16,), jnp.float32)
        return carry
    lax.fori_loop(0, rows, body, 0)


@functools.partial(jax.jit, static_argnames=("ch",))
def _degree(dst_p, ch):
    @functools.partial(
        pl.kernel,
        out_type=jax.ShapeDtypeStruct((NC, N, 16), jnp.float32),
        mesh=_mesh(),
        scratch_types=[
            pltpu.VMEM((ch, CHUNK), jnp.int32),
            pltpu.VMEM((CHUNK, 16), jnp.float32),
            pltpu.VMEM_SHARED((N_PAD, 16), jnp.float32),
        ],
    )
    def deg_kernel(dst_hbm, out_hbm, dst_v, ones_v, acc_sh):
        c = lax.axis_index("c")
        s = lax.axis_index("s")
        _zero_buf(ones_v, CHUNK, 16)
        zbase = s * ZROWS
        for r in range(ZROWS // CHUNK):
            pltpu.sync_copy(ones_v, acc_sh.at[pl.ds(zbase + r * CHUNK, CHUNK)])
        plsc.subcore_barrier()

        def fill_ones(i, carry):
            ones_v[i, :] = jnp.ones((16,), jnp.float32)
            return carry
        lax.fori_loop(0, CHUNK, fill_ones, 0)
        pltpu.sync_copy(dst_hbm.at[c, s], dst_v)

        def body(j, carry):
            pltpu.sync_copy(ones_v, acc_sh.at[dst_v.at[j]], add=True)
            return carry
        lax.fori_loop(0, ch, body, 0)
        plsc.subcore_barrier()
        dbase = s * DROWS
        pltpu.sync_copy(acc_sh.at[pl.ds(dbase, DROWS)],
                        out_hbm.at[c, pl.ds(dbase, DROWS)])

    return deg_kernel(dst_p)


@functools.partial(jax.jit, static_argnames=("ch",))
def _propagate(hs, src_p, dst_p, ch):
    @functools.partial(
        pl.kernel,
        out_type=jax.ShapeDtypeStruct((NC, N, D), jnp.float32),
        mesh=_mesh(),
        scratch_types=[
            pltpu.VMEM((ch, CHUNK), jnp.int32),
            pltpu.VMEM((ch, CHUNK), jnp.int32),
            pltpu.VMEM((CHUNK, D), jnp.float32),
            pltpu.VMEM_SHARED((N_PAD, D), jnp.float32),
            pltpu.SemaphoreType.DMA,
        ],
    )
    def prop_kernel(hs_hbm, src_hbm, dst_hbm, out_hbm,
                    src_v, dst_v, rows_v, acc_sh, sem):
        c = lax.axis_index("c")
        s = lax.axis_index("s")
        _zero_buf(rows_v, CHUNK, D)
        zbase = s * ZROWS
        for r in range(ZROWS // CHUNK):
            pltpu.sync_copy(rows_v, acc_sh.at[pl.ds(zbase + r * CHUNK, CHUNK)])
        plsc.subcore_barrier()
        pltpu.sync_copy(src_hbm.at[c, s], src_v)
        pltpu.sync_copy(dst_hbm.at[c, s], dst_v)

        def body(j, carry):
            pltpu.async_copy(hs_hbm.at[src_v.at[j]], rows_v, sem).wait()
            pltpu.sync_copy(rows_v, acc_sh.at[dst_v.at[j]], add=True)
            return carry
        lax.fori_loop(0, ch, body, 0)
        plsc.subcore_barrier()
        dbase = s * DROWS
        pltpu.sync_copy(acc_sh.at[pl.ds(dbase, DROWS)],
                        out_hbm.at[c, pl.ds(dbase, DROWS)])

    return prop_kernel(hs, src_p, dst_p)


def _fc1(x, w, b, degp):
    def body(x_ref, w_ref, b_ref, degp_ref, x0_ref, hs_ref, dis_ref):
        dp = degp_ref[...]
        deg = dp[0, :, 0:1] + dp[1, :, 0:1] + 1.0
        dis = lax.rsqrt(deg)
        h = lax.dot_general(x_ref[...], w_ref[...], (((1,), (1,)), ((), ())),
                            preferred_element_type=jnp.float32)
        h = jnp.maximum(h + b_ref[...], 0.0)
        x0_ref[...] = h
        hs_ref[...] = h * dis
        dis_ref[...] = dis

    return pl.pallas_call(
        body,
        out_shape=(jax.ShapeDtypeStruct((N, D), jnp.float32),
                   jax.ShapeDtypeStruct((N, D), jnp.float32),
                   jax.ShapeDtypeStruct((N, 1), jnp.float32)),
    )(x, w, b, degp)


def _layer(p, hs, x0, dis, wc, g, bb, beta, wf=None, bf=None):
    last = wf is not None

    def body(p_ref, hs_ref, x0_ref, dis_ref, w_ref, g_ref, bb_ref, *rest):
        pp = p_ref[...]
        dis_v = dis_ref[...]
        ax = (pp[0] + pp[1] + hs_ref[...]) * dis_v
        hp = (1.0 - ALPHA) * ax + ALPHA * x0_ref[...]
        t = (1.0 - beta) * hp + beta * lax.dot_general(
            hp, w_ref[...], (((1,), (0,)), ((), ())),
            preferred_element_type=jnp.float32)
        mu = jnp.mean(t, axis=0, keepdims=True)
        var = jnp.mean((t - mu) ** 2, axis=0, keepdims=True)
        h = (t - mu) * lax.rsqrt(var + EPS) * g_ref[...] + bb_ref[...]
        h = jnp.maximum(h, 0.0)
        if last:
            wf_ref, bf_ref, out_ref = rest
            out_ref[...] = lax.dot_general(
                h, wf_ref[...], (((1,), (1,)), ((), ())),
                preferred_element_type=jnp.float32) + bf_ref[...]
        else:
            (out_ref,) = rest
            out_ref[...] = h * dis_v

    if last:
        return pl.pallas_call(
            body,
            out_shape=jax.ShapeDtypeStruct((N, D_OUT), jnp.float32),
        )(p, hs, x0, dis, wc, g, bb, wf, bf)
    return pl.pallas_call(
        body,
        out_shape=jax.ShapeDtypeStruct((N, D), jnp.float32),
    )(p, hs, x0, dis, wc, g, bb)


def kernel(x, edge_index, W_fc1, b_fc1, W_c0, W_c1, W_c2,
           g0, bb0, g1, bb1, g2, bb2, W_fc2, b_fc2):
    e = edge_index.shape[1]
    ch = -(-e // (NC * NS * CHUNK))
    e_pad = NC * NS * ch * CHUNK
    src = edge_index[0].astype(jnp.int32)
    dst = edge_index[1].astype(jnp.int32)
    src_p = jnp.concatenate(
        [src, jnp.zeros((e_pad - e,), jnp.int32)]).reshape(NC, NS, ch, CHUNK)
    dst_p = jnp.concatenate(
        [dst, jnp.full((e_pad - e,), N, jnp.int32)]).reshape(NC, NS, ch, CHUNK)

    degp = _degree(dst_p, ch=ch)
    x0, hs, dis = _fc1(x, W_fc1, b_fc1.reshape(1, D), degp)

    wcs = (W_c0, W_c1, W_c2)
    gs = (g0, g1, g2)
    bbs = (bb0, bb1, bb2)
    out = None
    for i in range(3):
        p = _propagate(hs, src_p, dst_p, ch=ch)
        beta = math.log(THETA / (i + 1.0) + 1.0)
        if i < 2:
            hs = _layer(p, hs, x0, dis, wcs[i], gs[i].reshape(1, D),
                        bbs[i].reshape(1, D), beta)
        else:
            out = _layer(p, hs, x0, dis, wcs[i], gs[i].reshape(1, D),
                         bbs[i].reshape(1, D), beta,
                         wf=W_fc2, bf=b_fc2.reshape(1, D_OUT))
    return out


# speedup vs baseline: 10.5440x; 10.5440x over previous
import functools
import math

import jax
import jax.numpy as jnp
from jax import lax
from jax.experimental import pallas as pl
from jax.experimental.pallas import tpu as pltpu
from jax.experimental.pallas import tpu_sc as plsc

N = 10000
D = 128
D_OUT = 40
ALPHA = 0.1
THETA = 0.5
EPS = 1e-5

NC = 2
NS = 16
CHUNK = 128
N_PAD = 10240
ZROWS = N_PAD // NS


def _mesh():
    return plsc.VectorSubcoreMesh(core_axis_name="c", subcore_axis_name="s")


def _zero_buf(buf, rows, width):
    def body(i, carry):
        for l in range(width // 16):
            buf[i, pl.ds(l * 16, 16)] = jnp.zeros((16,), jnp.float32)
        return carry
    lax.fori_loop(0, rows, body, 0)


@functools.partial(jax.jit, static_argnames=("ch",))
def _degree(dst_p, ch):
    @functools.partial(
        pl.kernel,
        out_type=jax.ShapeDtypeStruct((NC, N_PAD, D), jnp.float32),
        mesh=_mesh(),
        scratch_types=[
            pltpu.VMEM((ch, CHUNK), jnp.int32),
            pltpu.VMEM((CHUNK, D), jnp.float32),
            pltpu.VMEM_SHARED((N_PAD, D), jnp.float32),
        ],
    )
    def deg_kernel(dst_hbm, out_hbm, dst_v, ones_v, acc_sh):
        c = lax.axis_index("c")
        s = lax.axis_index("s")
        _zero_buf(ones_v, CHUNK, D)
        zbase = s * ZROWS
        for r in range(ZROWS // CHUNK):
            pltpu.sync_copy(ones_v, acc_sh.at[pl.ds(zbase + r * CHUNK, CHUNK)])
        plsc.subcore_barrier()

        def fill_ones(i, carry):
            for l in range(D // 16):
                ones_v[i, pl.ds(l * 16, 16)] = jnp.ones((16,), jnp.float32)
            return carry
        lax.fori_loop(0, CHUNK, fill_ones, 0)
        pltpu.sync_copy(dst_hbm.at[c, s], dst_v)

        def body(j, carry):
            pltpu.sync_copy(ones_v, acc_sh.at[dst_v.at[j]], add=True)
            return carry
        lax.fori_loop(0, ch, body, 0)
        plsc.subcore_barrier()
        pltpu.sync_copy(acc_sh.at[pl.ds(zbase, ZROWS)],
                        out_hbm.at[c, pl.ds(zbase, ZROWS)])

    return deg_kernel(dst_p)


@functools.partial(jax.jit, static_argnames=("ch",))
def _propagate(hs, src_p, dst_p, ch):
    @functools.partial(
        pl.kernel,
        out_type=jax.ShapeDtypeStruct((NC, N_PAD, D), jnp.float32),
        mesh=_mesh(),
        scratch_types=[
            pltpu.VMEM((ch, CHUNK), jnp.int32),
            pltpu.VMEM((ch, CHUNK), jnp.int32),
            pltpu.VMEM((CHUNK, D), jnp.float32),
            pltpu.VMEM_SHARED((N_PAD, D), jnp.float32),
            pltpu.SemaphoreType.DMA,
        ],
    )
    def prop_kernel(hs_hbm, src_hbm, dst_hbm, out_hbm,
                    src_v, dst_v, rows_v, acc_sh, sem):
        c = lax.axis_index("c")
        s = lax.axis_index("s")
        _zero_buf(rows_v, CHUNK, D)
        zbase = s * ZROWS
        for r in range(ZROWS // CHUNK):
            pltpu.sync_copy(rows_v, acc_sh.at[pl.ds(zbase + r * CHUNK, CHUNK)])
        plsc.subcore_barrier()
        pltpu.sync_copy(src_hbm.at[c, s], src_v)
        pltpu.sync_copy(dst_hbm.at[c, s], dst_v)

        def body(j, carry):
            pltpu.async_copy(hs_hbm.at[src_v.at[j]], rows_v, sem).wait()
            pltpu.sync_copy(rows_v, acc_sh.at[dst_v.at[j]], add=True)
            return carry
        lax.fori_loop(0, ch, body, 0)
        plsc.subcore_barrier()
        pltpu.sync_copy(acc_sh.at[pl.ds(zbase, ZROWS)],
                        out_hbm.at[c, pl.ds(zbase, ZROWS)])

    return prop_kernel(hs, src_p, dst_p)


def _fc1(x, w, b, degp):
    def body(x_ref, w_ref, b_ref, degp_ref, x0_ref, hs_ref, dis_ref):
        dp = degp_ref[...]
        deg = dp[0, :N, 0:1] + dp[1, :N, 0:1] + 1.0
        dis = lax.rsqrt(deg)
        h = lax.dot_general(x_ref[...], w_ref[...], (((1,), (1,)), ((), ())),
                            preferred_element_type=jnp.float32)
        h = jnp.maximum(h + b_ref[...], 0.0)
        x0_ref[...] = h
        hs_ref[...] = h * dis
        dis_ref[...] = dis

    return pl.pallas_call(
        body,
        out_shape=(jax.ShapeDtypeStruct((N, D), jnp.float32),
                   jax.ShapeDtypeStruct((N, D), jnp.float32),
                   jax.ShapeDtypeStruct((N, 1), jnp.float32)),
    )(x, w, b, degp)


def _layer(p, hs, x0, dis, wc, g, bb, beta, wf=None, bf=None):
    last = wf is not None

    def body(p_ref, hs_ref, x0_ref, dis_ref, w_ref, g_ref, bb_ref, *rest):
        pp = p_ref[...]
        dis_v = dis_ref[...]
        ax = (pp[0, :N] + pp[1, :N] + hs_ref[...]) * dis_v
        hp = (1.0 - ALPHA) * ax + ALPHA * x0_ref[...]
        t = (1.0 - beta) * hp + beta * lax.dot_general(
            hp, w_ref[...], (((1,), (0,)), ((), ())),
            preferred_element_type=jnp.float32)
        mu = jnp.mean(t, axis=0, keepdims=True)
        var = jnp.mean((t - mu) ** 2, axis=0, keepdims=True)
        h = (t - mu) * lax.rsqrt(var + EPS) * g_ref[...] + bb_ref[...]
        h = jnp.maximum(h, 0.0)
        if last:
            wf_ref, bf_ref, out_ref = rest
            out_ref[...] = lax.dot_general(
                h, wf_ref[...], (((1,), (1,)), ((), ())),
                preferred_element_type=jnp.float32) + bf_ref[...]
        else:
            (out_ref,) = rest
            out_ref[...] = h * dis_v

    if last:
        return pl.pallas_call(
            body,
            out_shape=jax.ShapeDtypeStruct((N, D_OUT), jnp.float32),
        )(p, hs, x0, dis, wc, g, bb, wf, bf)
    return pl.pallas_call(
        body,
        out_shape=jax.ShapeDtypeStruct((N, D), jnp.float32),
    )(p, hs, x0, dis, wc, g, bb)


def kernel(x, edge_index, W_fc1, b_fc1, W_c0, W_c1, W_c2,
           g0, bb0, g1, bb1, g2, bb2, W_fc2, b_fc2):
    e = edge_index.shape[1]
    ch = -(-e // (NC * NS * CHUNK))
    e_pad = NC * NS * ch * CHUNK
    src = edge_index[0].astype(jnp.int32)
    dst = edge_index[1].astype(jnp.int32)
    src_p = jnp.concatenate(
        [src, jnp.zeros((e_pad - e,), jnp.int32)]).reshape(NC, NS, ch, CHUNK)
    dst_p = jnp.concatenate(
        [dst, jnp.full((e_pad - e,), N, jnp.int32)]).reshape(NC, NS, ch, CHUNK)

    degp = _degree(dst_p, ch=ch)
    x0, hs, dis = _fc1(x, W_fc1, b_fc1.reshape(1, D), degp)

    wcs = (W_c0, W_c1, W_c2)
    gs = (g0, g1, g2)
    bbs = (bb0, bb1, bb2)
    out = None
    for i in range(3):
        p = _propagate(hs, src_p, dst_p, ch=ch)
        beta = math.log(THETA / (i + 1.0) + 1.0)
        if i < 2:
            hs = _layer(p, hs, x0, dis, wcs[i], gs[i].reshape(1, D),
                        bbs[i].reshape(1, D), beta)
        else:
            out = _layer(p, hs, x0, dis, wcs[i], gs[i].reshape(1, D),
                         bbs[i].reshape(1, D), beta,
                         wf=W_fc2, bf=b_fc2.reshape(1, D_OUT))
    return out
